# baseline (speedup 1.0000x reference)
import functools

import jax
import jax.numpy as jnp
from jax import lax
from jax.experimental import pallas as pl
from jax.experimental.pallas import tpu as pltpu
from jax.experimental.pallas import tpu_sc as plsc

_TM = 128


def _routing_tables(eid, n_tok, tm, n_exp):
    t = n_tok // tm
    p_max = t + n_exp - 1
    tok = jnp.arange(n_tok, dtype=jnp.int32)
    keys = eid * jnp.int32(n_tok) + tok
    perm = jnp.sort(keys) % jnp.int32(n_tok)
    inv_perm = jnp.zeros((n_tok,), jnp.int32).at[perm].set(tok)

    counts = jnp.bincount(eid, length=n_exp).astype(jnp.int32)
    ends = jnp.cumsum(counts)
    starts = ends - counts
    nonempty = counts > 0
    first = jnp.where(nonempty, starts // tm, 0)
    last = jnp.where(nonempty, (ends - 1) // tm, 0)
    span = jnp.where(nonempty, last - first + 1, 0)
    pair_start = jnp.cumsum(span) - span
    total = pair_start[n_exp - 1] + span[n_exp - 1]

    p = jnp.arange(p_max, dtype=jnp.int32)
    e_raw = (jnp.searchsorted(pair_start, p, side="right") - 1).astype(jnp.int32)
    e_raw = jnp.clip(e_raw, 0, n_exp - 1)
    tile_raw = jnp.clip(first[e_raw] + (p - pair_start[e_raw]), 0, t - 1)
    pad = p >= total
    experts = jnp.where(pad, jnp.take(e_raw, total - 1), e_raw)
    tiles = jnp.where(pad, t - 1, tile_raw)
    lo_g = jnp.clip(starts[experts], tiles * tm, (tiles + 1) * tm)
    hi_g = jnp.clip(ends[experts], tiles * tm, (tiles + 1) * tm)
    lo = lo_g - tiles * tm
    hi = hi_g - tiles * tm
    prev = jnp.concatenate([jnp.full((1,), -1, jnp.int32), experts[:-1]])
    newload = (experts != prev).astype(jnp.int32)
    loadslot = (jnp.cumsum(newload) - 1) % 2
    return perm, inv_perm, tiles, experts, lo, hi, newload, loadslot


def _ffn_body(tiles_r, experts_r, lo_r, hi_r, newload_r, loadslot_r,
              x_ref, w1_hbm, b1_ref, w2_hbm, b2_ref, o_ref,
              w1_buf, w2_buf, sem1, sem2):
    i = pl.program_id(0)
    n = pl.num_programs(0)
    lo = lo_r[i]
    hi = hi_r[i]
    e = experts_r[i]
    slot = loadslot_r[i]

    @pl.when(i == 0)
    def _():
        pltpu.make_async_copy(w1_hbm.at[e], w1_buf.at[0], sem1.at[0]).start()
        pltpu.make_async_copy(w2_hbm.at[e], w2_buf.at[0], sem2.at[0]).start()

    @pl.when(jnp.logical_and(i + 1 < n, newload_r[jnp.minimum(i + 1, n - 1)] == 1))
    def _():
        ne = experts_r[jnp.minimum(i + 1, n - 1)]
        ns = 1 - slot
        pltpu.make_async_copy(w1_hbm.at[ne], w1_buf.at[ns], sem1.at[ns]).start()
        pltpu.make_async_copy(w2_hbm.at[ne], w2_buf.at[ns], sem2.at[ns]).start()

    @pl.when(newload_r[i] == 1)
    def _():
        pltpu.make_async_copy(w1_hbm.at[e], w1_buf.at[slot], sem1.at[slot]).wait()
        pltpu.make_async_copy(w2_hbm.at[e], w2_buf.at[slot], sem2.at[slot]).wait()

    @pl.when(lo < hi)
    def _():
        xb = x_ref[...]
        nt = (((1,), (1,)), ((), ()))
        w1 = w1_buf[slot]
        h = lax.dot_general(xb, w1, nt, preferred_element_type=jnp.float32)
        h = jnp.maximum(h + b1_ref[0], 0.0)
        w2 = w2_buf[slot]
        y = lax.dot_general(h, w2, nt, preferred_element_type=jnp.float32)
        y = y + b2_ref[0]
        rows = lax.broadcasted_iota(jnp.int32, (xb.shape[0], 1), 0)
        m = (rows >= lo) & (rows < hi)
        o_ref[...] = jnp.where(m, y, o_ref[...])


def _grouped_ffn(x_sorted, w1, b1r, w2, b2r, tiles, experts, lo, hi,
                 newload, loadslot):
    n_tok, in_dim = x_sorted.shape
    n_exp, hid, _ = w1.shape
    out_dim = w2.shape[1]
    p_max = tiles.shape[0]
    grid_spec = pltpu.PrefetchScalarGridSpec(
        num_scalar_prefetch=6,
        grid=(p_max,),
        in_specs=[
            pl.BlockSpec((_TM, in_dim),
                         lambda i, tr, er, lr, hr, nl, ls: (tr[i], 0)),
            pl.BlockSpec(memory_space=pl.ANY),
            pl.BlockSpec((1, 1, hid),
                         lambda i, tr, er, lr, hr, nl, ls: (er[i], 0, 0)),
            pl.BlockSpec(memory_space=pl.ANY),
            pl.BlockSpec((1, 1, out_dim),
                         lambda i, tr, er, lr, hr, nl, ls: (er[i], 0, 0)),
        ],
        out_specs=pl.BlockSpec((_TM, out_dim),
                               lambda i, tr, er, lr, hr, nl, ls: (tr[i], 0)),
        scratch_shapes=[
            pltpu.VMEM((2, hid, in_dim), jnp.float32),
            pltpu.VMEM((2, out_dim, hid), jnp.float32),
            pltpu.SemaphoreType.DMA((2,)),
            pltpu.SemaphoreType.DMA((2,)),
        ],
    )
    return pl.pallas_call(
        _ffn_body,
        grid_spec=grid_spec,
        out_shape=jax.ShapeDtypeStruct((n_tok, out_dim), jnp.float32),
    )(tiles, experts, lo, hi, newload, loadslot, x_sorted, w1, b1r, w2, b2r)


@functools.lru_cache(maxsize=None)
def _make_sc_gather(n_rows, d):
    info = plsc.get_sparse_core_info()
    nc, ns = info.num_cores, info.num_subcores
    nw = nc * ns
    bpw = n_rows // nw
    mesh = plsc.VectorSubcoreMesh(core_axis_name="c", subcore_axis_name="s")

    @functools.partial(
        pl.kernel, mesh=mesh,
        out_type=jax.ShapeDtypeStruct((n_rows, d), jnp.float32),
        scratch_types=[
            pltpu.VMEM((bpw,), jnp.int32),
            pltpu.VMEM((bpw, d), jnp.float32),
            pltpu.SemaphoreType.DMA,
        ],
    )
    def gather_k(table_hbm, idx_hbm, out_hbm, idx_v, rows_v, sem):
        wid = lax.axis_index("s") * nc + lax.axis_index("c")
        base = wid * bpw
        pltpu.sync_copy(idx_hbm.at[pl.ds(base, bpw)], idx_v)
        pltpu.async_copy(table_hbm.at[idx_v], rows_v, sem).wait()
        pltpu.sync_copy(rows_v, out_hbm.at[pl.ds(base, bpw)])

    return gather_k


def kernel(x, Wg, bg, W1, b1, W2, b2):
    n_tok, in_dim = x.shape
    n_exp, hid, _ = W1.shape
    out_dim = W2.shape[1]

    gating_scores = x @ Wg.T + bg
    _, top_idx = lax.top_k(gating_scores, 1)
    eid = top_idx[:, 0].astype(jnp.int32)

    perm, inv_perm, tiles, experts, lo, hi, newload, loadslot = \
        _routing_tables(eid, n_tok, _TM, n_exp)

    gather = _make_sc_gather(n_tok, in_dim)
    x_sorted = gather(x, perm)

    y_sorted = _grouped_ffn(
        x_sorted, W1, b1.reshape(n_exp, 1, hid), W2,
        b2.reshape(n_exp, 1, out_dim), tiles, experts, lo, hi,
        newload, loadslot)

    combine = _make_sc_gather(n_tok, out_dim)
    return combine(y_sorted, inv_perm)


# speedup vs baseline: 1.1065x; 1.1065x over previous
import functools

import jax
import jax.numpy as jnp
from jax import lax
from jax.experimental import pallas as pl
from jax.experimental.pallas import tpu as pltpu
from jax.experimental.pallas import tpu_sc as plsc

_TM = 128


def _routing_tables(eid, n_tok, tm, n_exp):
    t = n_tok // tm
    p_max = t + n_exp - 1
    tok = jnp.arange(n_tok, dtype=jnp.int32)
    keys = eid * jnp.int32(n_tok) + tok
    perm = jnp.sort(keys) % jnp.int32(n_tok)
    inv_perm = jnp.zeros((n_tok,), jnp.int32).at[perm].set(tok)

    counts = jnp.bincount(eid, length=n_exp).astype(jnp.int32)
    ends = jnp.cumsum(counts)
    starts = ends - counts
    nonempty = counts > 0
    first = jnp.where(nonempty, starts // tm, 0)
    last = jnp.where(nonempty, (ends - 1) // tm, 0)
    span = jnp.where(nonempty, last - first + 1, 0)
    pair_start = jnp.cumsum(span) - span
    total = pair_start[n_exp - 1] + span[n_exp - 1]

    p = jnp.arange(p_max, dtype=jnp.int32)
    e_raw = (jnp.searchsorted(pair_start, p, side="right") - 1).astype(jnp.int32)
    e_raw = jnp.clip(e_raw, 0, n_exp - 1)
    tile_raw = jnp.clip(first[e_raw] + (p - pair_start[e_raw]), 0, t - 1)
    pad = p >= total
    experts = jnp.where(pad, jnp.take(e_raw, total - 1), e_raw)
    tiles = jnp.where(pad, t - 1, tile_raw)
    lo_g = jnp.clip(starts[experts], tiles * tm, (tiles + 1) * tm)
    hi_g = jnp.clip(ends[experts], tiles * tm, (tiles + 1) * tm)
    lo = lo_g - tiles * tm
    hi = hi_g - tiles * tm
    prev = jnp.concatenate([jnp.full((1,), -1, jnp.int32), experts[:-1]])
    newload = (experts != prev).astype(jnp.int32)
    loadidx = jnp.cumsum(newload) - 1
    load_expert = jnp.zeros((n_exp,), jnp.int32).at[loadidx].set(experts)
    n_loads = jnp.broadcast_to(loadidx[-1] + 1, (1,)).astype(jnp.int32)
    return (perm, inv_perm, tiles, experts, lo, hi, newload, loadidx,
            load_expert, n_loads)


_NSLOT = 3


def _issue_load(l, w1_hbm, w2_hbm, w1_buf, w2_buf, sem1, sem2,
                load_expert_r, n_loads_r):
    @pl.when(l < n_loads_r[0])
    def _():
        le = load_expert_r[jnp.minimum(l, load_expert_r.shape[0] - 1)]
        s = lax.rem(l, _NSLOT)
        pltpu.make_async_copy(w1_hbm.at[le], w1_buf.at[s], sem1.at[s]).start()
        pltpu.make_async_copy(w2_hbm.at[le], w2_buf.at[s], sem2.at[s]).start()


def _ffn_body(tiles_r, experts_r, lo_r, hi_r, newload_r, loadidx_r,
              load_expert_r, n_loads_r,
              x_ref, w1_hbm, b1_ref, w2_hbm, b2_ref, o_ref,
              w1_buf, w2_buf, sem1, sem2):
    i = pl.program_id(0)
    lo = lo_r[i]
    hi = hi_r[i]
    e = experts_r[i]
    l = loadidx_r[i]
    slot = lax.rem(l, _NSLOT)
    issue = functools.partial(
        _issue_load, w1_hbm=w1_hbm, w2_hbm=w2_hbm, w1_buf=w1_buf,
        w2_buf=w2_buf, sem1=sem1, sem2=sem2, load_expert_r=load_expert_r,
        n_loads_r=n_loads_r)

    @pl.when(i == 0)
    def _():
        issue(jnp.int32(0))
        issue(jnp.int32(1))

    @pl.when(newload_r[i] == 1)
    def _():
        issue(l + 2)
        pltpu.make_async_copy(w1_hbm.at[e], w1_buf.at[slot],
                              sem1.at[slot]).wait()

    @pl.when(lo < hi)
    def _():
        xb = x_ref[...]
        nt = (((1,), (1,)), ((), ()))
        w1 = w1_buf[slot]
        h = lax.dot_general(xb, w1, nt, preferred_element_type=jnp.float32)
        h = jnp.maximum(h + b1_ref[0], 0.0)

        @pl.when(newload_r[i] == 1)
        def _():
            pltpu.make_async_copy(w2_hbm.at[e], w2_buf.at[slot],
                                  sem2.at[slot]).wait()

        w2 = w2_buf[slot]
        y = lax.dot_general(h, w2, nt, preferred_element_type=jnp.float32)
        y = y + b2_ref[0]
        rows = lax.broadcasted_iota(jnp.int32, (xb.shape[0], 1), 0)
        m = (rows >= lo) & (rows < hi)
        o_ref[...] = jnp.where(m, y, o_ref[...])


def _grouped_ffn(x_sorted, w1, b1r, w2, b2r, tiles, experts, lo, hi,
                 newload, loadidx, load_expert, n_loads):
    n_tok, in_dim = x_sorted.shape
    n_exp, hid, _ = w1.shape
    out_dim = w2.shape[1]
    p_max = tiles.shape[0]
    grid_spec = pltpu.PrefetchScalarGridSpec(
        num_scalar_prefetch=8,
        grid=(p_max,),
        in_specs=[
            pl.BlockSpec((_TM, in_dim),
                         lambda i, tr, er, lr, hr, nl, li, le, nn: (tr[i], 0)),
            pl.BlockSpec(memory_space=pl.ANY),
            pl.BlockSpec((1, 1, hid),
                         lambda i, tr, er, lr, hr, nl, li, le, nn: (er[i], 0, 0)),
            pl.BlockSpec(memory_space=pl.ANY),
            pl.BlockSpec((1, 1, out_dim),
                         lambda i, tr, er, lr, hr, nl, li, le, nn: (er[i], 0, 0)),
        ],
        out_specs=pl.BlockSpec((_TM, out_dim),
                               lambda i, tr, er, lr, hr, nl, li, le, nn: (tr[i], 0)),
        scratch_shapes=[
            pltpu.VMEM((_NSLOT, hid, in_dim), jnp.float32),
            pltpu.VMEM((_NSLOT, out_dim, hid), jnp.float32),
            pltpu.SemaphoreType.DMA((_NSLOT,)),
            pltpu.SemaphoreType.DMA((_NSLOT,)),
        ],
    )
    return pl.pallas_call(
        _ffn_body,
        grid_spec=grid_spec,
        out_shape=jax.ShapeDtypeStruct((n_tok, out_dim), jnp.float32),
    )(tiles, experts, lo, hi, newload, loadidx, load_expert, n_loads,
      x_sorted, w1, b1r, w2, b2r)


@functools.lru_cache(maxsize=None)
def _make_sc_gather(n_rows, d):
    info = plsc.get_sparse_core_info()
    nc, ns = info.num_cores, info.num_subcores
    nw = nc * ns
    bpw = n_rows // nw
    mesh = plsc.VectorSubcoreMesh(core_axis_name="c", subcore_axis_name="s")

    @functools.partial(
        pl.kernel, mesh=mesh,
        out_type=jax.ShapeDtypeStruct((n_rows, d), jnp.float32),
        scratch_types=[
            pltpu.VMEM((bpw,), jnp.int32),
            pltpu.VMEM((bpw, d), jnp.float32),
            pltpu.SemaphoreType.DMA,
        ],
    )
    def gather_k(table_hbm, idx_hbm, out_hbm, idx_v, rows_v, sem):
        wid = lax.axis_index("s") * nc + lax.axis_index("c")
        base = wid * bpw
        pltpu.sync_copy(idx_hbm.at[pl.ds(base, bpw)], idx_v)
        pltpu.async_copy(table_hbm.at[idx_v], rows_v, sem).wait()
        pltpu.sync_copy(rows_v, out_hbm.at[pl.ds(base, bpw)])

    return gather_k


def kernel(x, Wg, bg, W1, b1, W2, b2):
    n_tok, in_dim = x.shape
    n_exp, hid, _ = W1.shape
    out_dim = W2.shape[1]

    gating_scores = x @ Wg.T + bg
    _, top_idx = lax.top_k(gating_scores, 1)
    eid = top_idx[:, 0].astype(jnp.int32)

    (perm, inv_perm, tiles, experts, lo, hi, newload, loadidx,
     load_expert, n_loads) = _routing_tables(eid, n_tok, _TM, n_exp)

    gather = _make_sc_gather(n_tok, in_dim)
    x_sorted = gather(x, perm)

    y_sorted = _grouped_ffn(
        x_sorted, W1, b1.reshape(n_exp, 1, hid), W2,
        b2.reshape(n_exp, 1, out_dim), tiles, experts, lo, hi,
        newload, loadidx, load_expert, n_loads)

    combine = _make_sc_gather(n_tok, out_dim)
    return combine(y_sorted, inv_perm)
